# baseline (speedup 1.0000x reference)
import functools
import jax
import jax.numpy as jnp
from jax import lax
from jax.experimental import pallas as pl
from jax.experimental.pallas import tpu as pltpu
from jax.experimental.pallas import tpu_sc as plsc

N_PTS = 16384
N_BATCH = 4
N = N_PTS // N_BATCH
M = N // 4
K = 32
F32 = jnp.float32
I32 = jnp.int32


def _two_sum(a, b):
    s = a + b
    bb = s - a
    err = (a - (s - bb)) + (b - bb)
    return s, err


def _fast_two_sum(a, b):
    s = a + b
    return s, b - (s - a)


def _split(a):
    c = a * F32(4097.0)
    hi = c - (c - a)
    return hi, a - hi


def _two_prod(a, b):
    p = a * b
    ahi, alo = _split(a)
    bhi, blo = _split(b)
    err = ((ahi * bhi - p) + ahi * blo + alo * bhi) + alo * blo
    return p, err


def _dd_add(ah, al, bh, bl):
    s, e = _two_sum(ah, bh)
    e = e + (al + bl)
    return _fast_two_sum(s, e)


def _dd_sqdiff(a, b):
    dh, dl = _two_sum(a, -b)
    p = dh * dh
    hi, lo = _split(dh)
    pe = ((hi * hi - p) + F32(2.0) * (hi * lo)) + lo * lo
    pe = pe + F32(2.0) * (dh * dl)
    return _fast_two_sum(p, pe)


def _dd_sqdist(px, py, pz, qx, qy, qz):
    sxh, sxl = _dd_sqdiff(px, qx)
    syh, syl = _dd_sqdiff(py, qy)
    szh, szl = _dd_sqdiff(pz, qz)
    h, l = _dd_add(sxh, sxl, syh, syl)
    return _dd_add(h, l, szh, szl)


R2 = 2 * N_BATCH
NH = N // 2


def _fps_body(posT_ref, out_ref):
    px = posT_ref[0]
    py = posT_ref[1]
    pz = posT_ref[2]
    giota = (lax.broadcasted_iota(I32, (R2, NH), 1)
             + (lax.broadcasted_iota(I32, (R2, NH), 0) % 2) * NH)
    even = lax.broadcasted_iota(I32, (R2, 1), 0) % 2 == 0

    def pair(a, red2):
        up = jnp.concatenate([a[1:], a[:1]], axis=0)
        dn = jnp.concatenate([a[-1:], a[:-1]], axis=0)
        return red2(a, jnp.where(even, up, dn))

    def store_row(i, qx, qy, qz):
        out_ref[0, pl.ds(i, 1), :] = qx[:, 0][None, :]
        out_ref[1, pl.ds(i, 1), :] = qy[:, 0][None, :]
        out_ref[2, pl.ds(i, 1), :] = qz[:, 0][None, :]

    def bcast0(p):
        a = p[:, 0:1]
        sh = jnp.concatenate([a[:1], a[:-1]], axis=0)
        return jnp.where(even, a, sh)

    qx = bcast0(px)
    qy = bcast0(py)
    qz = bcast0(pz)
    store_row(0, qx, qy, qz)
    dh, dl = _dd_sqdist(px, py, pz, qx, qy, qz)

    def body(i, carry):
        dh, dl = carry
        mh = pair(jnp.max(dh, axis=1, keepdims=True), jnp.maximum)
        eqh = dh == mh
        ml = pair(jnp.max(jnp.where(eqh, dl, -jnp.inf), axis=1, keepdims=True),
                  jnp.maximum)
        cand = eqh & (dl == ml)
        j = pair(jnp.min(jnp.where(cand, giota, I32(N)), axis=1, keepdims=True),
                 jnp.minimum)
        msk = (giota == j).astype(F32)
        qx = pair(jnp.sum(px * msk, axis=1, keepdims=True), jnp.add)
        qy = pair(jnp.sum(py * msk, axis=1, keepdims=True), jnp.add)
        qz = pair(jnp.sum(pz * msk, axis=1, keepdims=True), jnp.add)
        store_row(i, qx, qy, qz)
        nh, nl = _dd_sqdist(px, py, pz, qx, qy, qz)
        take = (nh < dh) | ((nh == dh) & (nl < dl))
        return jnp.where(take, nh, dh), jnp.where(take, nl, dl)

    lax.fori_loop(1, M, body, (dh, dl))


def _fps_call(posT8):
    return pl.pallas_call(
        _fps_body,
        out_shape=jax.ShapeDtypeStruct((3, M, R2), F32),
    )(posT8)


def _sample_positions(pos):
    posT8 = jnp.transpose(pos.reshape(R2, NH, 3), (2, 0, 1))
    pd = _fps_call(posT8)
    return jnp.transpose(pd[:, :, 0::2], (2, 1, 0)).reshape(N_BATCH * M, 3)


Q_TILE = 256


def _knn_body(q_ref, posT_ref, col_ref):
    b = pl.program_id(0)
    q = q_ref[...]
    qx = q[:, 0:1]
    qy = q[:, 1:2]
    qz = q[:, 2:3]
    s = posT_ref[0]
    sx = s[0:1, :]
    sy = s[1:2, :]
    sz = s[2:3, :]
    dx = qx - sx
    dy = qy - sy
    dz = qz - sz
    d = dx * dx
    d = d + dy * dy
    d = d + dz * dz
    iota = lax.broadcasted_iota(I32, (Q_TILE, N), 1)
    inf = F32(jnp.inf)
    for t in range(K):
        m = jnp.min(d, axis=1, keepdims=True)
        sel = d == m
        j = jnp.min(jnp.where(sel, iota, I32(N)), axis=1, keepdims=True)
        col_ref[:, t:t + 1] = j
        d = jnp.where(iota == j, inf, d)


def _knn_call(pos_dst, posTB):
    return pl.pallas_call(
        _knn_body,
        grid=(N_BATCH, M // Q_TILE),
        in_specs=[
            pl.BlockSpec((Q_TILE, 3), lambda b, t: (b * (M // Q_TILE) + t, I32(0))),
            pl.BlockSpec((1, 3, N), lambda b, t: (b, I32(0), I32(0))),
        ],
        out_specs=pl.BlockSpec((Q_TILE, K), lambda b, t: (b * (M // Q_TILE) + t, I32(0))),
        out_shape=jax.ShapeDtypeStruct((N_BATCH * M, K), I32),
    )(pos_dst, posTB)


D_TBL = 8
N_EDGE = N_BATCH * M * K
HALF = 2048


def _sc_gather(feats, lcol):
    info = plsc.get_sparse_core_info()
    nc, ns = info.num_cores, info.num_subcores
    nw = nc * ns
    e_per_w = N_EDGE // nw

    @functools.partial(
        pl.kernel,
        mesh=plsc.VectorSubcoreMesh(core_axis_name="c", subcore_axis_name="s"),
        out_type=jax.ShapeDtypeStruct((N_EDGE * D_TBL,), F32),
        compiler_params=pltpu.CompilerParams(needs_layout_passes=False),
        scratch_types=[
            pltpu.VMEM((6 * N,), F32),
            pltpu.VMEM((e_per_w,), I32),
            pltpu.VMEM((HALF * D_TBL,), F32),
        ],
    )
    def gather_k(feats_hbm, lcol_hbm, out_hbm, tbl_v, lidx_v, rows_v):
        wid = lax.axis_index("s") * nc + lax.axis_index("c")
        b = wid // (nw // N_BATCH)
        e0 = wid * e_per_w
        for c in range(6):
            pltpu.sync_copy(feats_hbm.at[pl.ds(c * N_PTS + b * N, N)],
                            tbl_v.at[pl.ds(c * N, N)])
        pltpu.sync_copy(lcol_hbm.at[pl.ds(e0, e_per_w)], lidx_v)
        lane = lax.iota(I32, 16)

        def do_half(half, _):
            def grp(g, _):
                iv = lidx_v[pl.ds(half * I32(HALF) + g * I32(16), 16)]
                rows = (g * I32(16) + lane) * I32(D_TBL)
                for c in range(6):
                    v = plsc.load_gather(tbl_v, [iv + I32(c * N)])
                    plsc.store_scatter(rows_v, [rows + I32(c)], v)
                return I32(0)
            lax.fori_loop(I32(0), I32(HALF // 16), grp, I32(0))
            pltpu.sync_copy(
                rows_v,
                out_hbm.at[pl.ds((e0 + half * I32(HALF)) * I32(D_TBL), HALF * D_TBL)])
            return I32(0)

        lax.fori_loop(I32(0), I32(e_per_w // HALF), do_half, I32(0))

    return gather_k(feats, lcol)


E_TILE = 2048
Q4_TILE = E_TILE // K


def _mlp_body(e_ref, q_ref, w1_ref, b1_ref, w2_ref, b2_ref, w3_ref, b3_ref, out_ref):
    hi = lax.Precision.HIGHEST
    t = e_ref[...][:, 0:6]
    h = jnp.dot(t, w1_ref[...], precision=hi, preferred_element_type=F32)
    pq = q_ref[...]
    c = jnp.dot(pq, w1_ref[3:6, :], precision=hi, preferred_element_type=F32)
    h3 = h.reshape(Q4_TILE, K, 64) + (b1_ref[...] - c)[:, None, :]
    h = jnp.maximum(h3, F32(0.0)).reshape(E_TILE, 64)
    h = jnp.dot(h, w2_ref[...], precision=hi, preferred_element_type=F32) + b2_ref[...]
    h = jnp.maximum(h, F32(0.0))
    h = jnp.dot(h, w3_ref[...], precision=hi, preferred_element_type=F32) + b3_ref[...]
    out_ref[...] = jnp.max(h.reshape(Q4_TILE, K, 128), axis=1)


def _mlp_call(edges, pos_dst, w1p, b1, w2, b2, w3, b3):
    n_tile = N_EDGE // E_TILE
    zero2 = lambda g: (I32(0), I32(0))
    return pl.pallas_call(
        _mlp_body,
        grid=(n_tile,),
        in_specs=[
            pl.BlockSpec((E_TILE, D_TBL), lambda g: (g, I32(0))),
            pl.BlockSpec((Q4_TILE, 3), lambda g: (g, I32(0))),
            pl.BlockSpec((6, 64), zero2),
            pl.BlockSpec((1, 64), zero2),
            pl.BlockSpec((64, 64), zero2),
            pl.BlockSpec((1, 64), zero2),
            pl.BlockSpec((64, 128), zero2),
            pl.BlockSpec((1, 128), zero2),
        ],
        out_specs=pl.BlockSpec((Q4_TILE, 128), lambda g: (g, I32(0))),
        out_shape=jax.ShapeDtypeStruct((N_BATCH * M, 128), F32),
    )(edges, pos_dst, w1p, b1, w2, b2, w3, b3)


def kernel(x, pos, batch, p0, p1, p2, p3, p4, p5):
    x = x.astype(F32)
    pos = pos.astype(F32)
    pos_dst = _sample_positions(pos)
    posTB = jnp.transpose(pos.reshape(N_BATCH, N, 3), (0, 2, 1))
    col = _knn_call(pos_dst, posTB).reshape(-1)
    feats = jnp.concatenate([x.T, pos.T], axis=0).reshape(-1)
    edges = _sc_gather(feats, col).reshape(N_EDGE, D_TBL)
    out = _mlp_call(edges, pos_dst, p0.astype(F32),
                    p1.reshape(1, 64).astype(F32), p2.astype(F32),
                    p3.reshape(1, 64).astype(F32), p4.astype(F32),
                    p5.reshape(1, 128).astype(F32))
    batch_dst = batch.reshape(N_BATCH, N)[:, :M].reshape(-1)
    return out, pos_dst, batch_dst


# speedup vs baseline: 1.1255x; 1.1255x over previous
import functools
import jax
import jax.numpy as jnp
from jax import lax
from jax.experimental import pallas as pl
from jax.experimental.pallas import tpu as pltpu
from jax.experimental.pallas import tpu_sc as plsc

N_PTS = 16384
N_BATCH = 4
N = N_PTS // N_BATCH
M = N // 4
K = 32
F32 = jnp.float32
I32 = jnp.int32


def _two_sum(a, b):
    s = a + b
    bb = s - a
    err = (a - (s - bb)) + (b - bb)
    return s, err


def _fast_two_sum(a, b):
    s = a + b
    return s, b - (s - a)


def _split(a):
    c = a * F32(4097.0)
    hi = c - (c - a)
    return hi, a - hi


def _two_prod(a, b):
    p = a * b
    ahi, alo = _split(a)
    bhi, blo = _split(b)
    err = ((ahi * bhi - p) + ahi * blo + alo * bhi) + alo * blo
    return p, err


def _dd_add(ah, al, bh, bl):
    s, e = _two_sum(ah, bh)
    e = e + (al + bl)
    return _fast_two_sum(s, e)


def _dd_sqdiff(a, b):
    dh, dl = _two_sum(a, -b)
    p = dh * dh
    hi, lo = _split(dh)
    pe = ((hi * hi - p) + F32(2.0) * (hi * lo)) + lo * lo
    pe = pe + F32(2.0) * (dh * dl)
    return _fast_two_sum(p, pe)


def _dd_sqdist(px, py, pz, qx, qy, qz):
    sxh, sxl = _dd_sqdiff(px, qx)
    syh, syl = _dd_sqdiff(py, qy)
    szh, szl = _dd_sqdiff(pz, qz)
    h, l = _dd_add(sxh, sxl, syh, syl)
    return _dd_add(h, l, szh, szl)


R2 = 2 * N_BATCH
NH = N // 2


def _fps_body(posT_ref, out_ref):
    px = posT_ref[0]
    py = posT_ref[1]
    pz = posT_ref[2]
    iota = lax.broadcasted_iota(I32, (N_BATCH, N), 1)

    def store_row(i, qx, qy, qz):
        out_ref[0, pl.ds(i, 1), :] = qx[:, 0][None, :]
        out_ref[1, pl.ds(i, 1), :] = qy[:, 0][None, :]
        out_ref[2, pl.ds(i, 1), :] = qz[:, 0][None, :]

    qx = px[:, 0:1]
    qy = py[:, 0:1]
    qz = pz[:, 0:1]
    store_row(0, qx, qy, qz)
    dh, dl = _dd_sqdist(px, py, pz, qx, qy, qz)

    def body(i, carry):
        dh, dl = carry
        mh = jnp.max(dh, axis=1, keepdims=True)
        eqh = dh == mh
        ml = jnp.max(jnp.where(eqh, dl, -jnp.inf), axis=1, keepdims=True)
        cand = eqh & (dl == ml)
        j = jnp.min(jnp.where(cand, iota, I32(N)), axis=1, keepdims=True)
        msk = (iota == j).astype(F32)
        qx = jnp.sum(px * msk, axis=1, keepdims=True)
        qy = jnp.sum(py * msk, axis=1, keepdims=True)
        qz = jnp.sum(pz * msk, axis=1, keepdims=True)
        store_row(i, qx, qy, qz)
        nh, nl = _dd_sqdist(px, py, pz, qx, qy, qz)
        take = (nh < dh) | ((nh == dh) & (nl < dl))
        return jnp.where(take, nh, dh), jnp.where(take, nl, dl)

    lax.fori_loop(1, M, body, (dh, dl))


def _fps_call(posT):
    return pl.pallas_call(
        _fps_body,
        out_shape=jax.ShapeDtypeStruct((3, M, N_BATCH), F32),
    )(posT)


def _sample_positions(pos):
    posT = jnp.transpose(pos.reshape(N_BATCH, N, 3), (2, 0, 1))
    pd = _fps_call(posT)
    return jnp.transpose(pd, (2, 1, 0)).reshape(N_BATCH * M, 3)


Q_TILE = 256


def _knn_body(q_ref, posT_ref, col_ref):
    b = pl.program_id(0)
    q = q_ref[...]
    qx = q[:, 0:1]
    qy = q[:, 1:2]
    qz = q[:, 2:3]
    s = posT_ref[0]
    sx = s[0:1, :]
    sy = s[1:2, :]
    sz = s[2:3, :]
    dx = qx - sx
    dy = qy - sy
    dz = qz - sz
    d = dx * dx
    d = d + dy * dy
    d = d + dz * dz
    iota = lax.broadcasted_iota(I32, (Q_TILE, N), 1)
    inf = F32(jnp.inf)
    for t in range(K):
        m = jnp.min(d, axis=1, keepdims=True)
        sel = d == m
        j = jnp.min(jnp.where(sel, iota, I32(N)), axis=1, keepdims=True)
        col_ref[:, t:t + 1] = j
        d = jnp.where(iota == j, inf, d)


def _knn_call(pos_dst, posTB):
    return pl.pallas_call(
        _knn_body,
        grid=(N_BATCH, M // Q_TILE),
        in_specs=[
            pl.BlockSpec((Q_TILE, 3), lambda b, t: (b * (M // Q_TILE) + t, I32(0))),
            pl.BlockSpec((1, 3, N), lambda b, t: (b, I32(0), I32(0))),
        ],
        out_specs=pl.BlockSpec((Q_TILE, K), lambda b, t: (b * (M // Q_TILE) + t, I32(0))),
        out_shape=jax.ShapeDtypeStruct((N_BATCH * M, K), I32),
    )(pos_dst, posTB)


D_TBL = 8
N_EDGE = N_BATCH * M * K
HALF = 2048


def _sc_gather(feats, lcol):
    info = plsc.get_sparse_core_info()
    nc, ns = info.num_cores, info.num_subcores
    nw = nc * ns
    e_per_w = N_EDGE // nw

    @functools.partial(
        pl.kernel,
        mesh=plsc.VectorSubcoreMesh(core_axis_name="c", subcore_axis_name="s"),
        out_type=jax.ShapeDtypeStruct((N_EDGE * D_TBL,), F32),
        compiler_params=pltpu.CompilerParams(needs_layout_passes=False),
        scratch_types=[
            pltpu.VMEM((6 * N,), F32),
            pltpu.VMEM((e_per_w,), I32),
            pltpu.VMEM((HALF * D_TBL,), F32),
        ],
    )
    def gather_k(feats_hbm, lcol_hbm, out_hbm, tbl_v, lidx_v, rows_v):
        wid = lax.axis_index("s") * nc + lax.axis_index("c")
        b = wid // (nw // N_BATCH)
        e0 = wid * e_per_w
        for c in range(6):
            pltpu.sync_copy(feats_hbm.at[pl.ds(c * N_PTS + b * N, N)],
                            tbl_v.at[pl.ds(c * N, N)])
        pltpu.sync_copy(lcol_hbm.at[pl.ds(e0, e_per_w)], lidx_v)
        lane = lax.iota(I32, 16)

        def do_half(half, _):
            def grp(g, _):
                iv = lidx_v[pl.ds(half * I32(HALF) + g * I32(16), 16)]
                rows = (g * I32(16) + lane) * I32(D_TBL)
                for c in range(6):
                    v = plsc.load_gather(tbl_v, [iv + I32(c * N)])
                    plsc.store_scatter(rows_v, [rows + I32(c)], v)
                return I32(0)
            lax.fori_loop(I32(0), I32(HALF // 16), grp, I32(0))
            pltpu.sync_copy(
                rows_v,
                out_hbm.at[pl.ds((e0 + half * I32(HALF)) * I32(D_TBL), HALF * D_TBL)])
            return I32(0)

        lax.fori_loop(I32(0), I32(e_per_w // HALF), do_half, I32(0))

    return gather_k(feats, lcol)


E_TILE = 2048
Q4_TILE = E_TILE // K


def _mlp_body(e_ref, q_ref, w1_ref, b1_ref, w2_ref, b2_ref, w3_ref, b3_ref, out_ref):
    hi = lax.Precision.HIGHEST
    t = e_ref[...][:, 0:6]
    h = jnp.dot(t, w1_ref[...], precision=hi, preferred_element_type=F32)
    pq = q_ref[...]
    c = jnp.dot(pq, w1_ref[3:6, :], precision=hi, preferred_element_type=F32)
    h3 = h.reshape(Q4_TILE, K, 64) + (b1_ref[...] - c)[:, None, :]
    h = jnp.maximum(h3, F32(0.0)).reshape(E_TILE, 64)
    h = jnp.dot(h, w2_ref[...], precision=hi, preferred_element_type=F32) + b2_ref[...]
    h = jnp.maximum(h, F32(0.0))
    h = jnp.dot(h, w3_ref[...], precision=hi, preferred_element_type=F32) + b3_ref[...]
    out_ref[...] = jnp.max(h.reshape(Q4_TILE, K, 128), axis=1)


def _mlp_call(edges, pos_dst, w1p, b1, w2, b2, w3, b3):
    n_tile = N_EDGE // E_TILE
    zero2 = lambda g: (I32(0), I32(0))
    return pl.pallas_call(
        _mlp_body,
        grid=(n_tile,),
        in_specs=[
            pl.BlockSpec((E_TILE, D_TBL), lambda g: (g, I32(0))),
            pl.BlockSpec((Q4_TILE, 3), lambda g: (g, I32(0))),
            pl.BlockSpec((6, 64), zero2),
            pl.BlockSpec((1, 64), zero2),
            pl.BlockSpec((64, 64), zero2),
            pl.BlockSpec((1, 64), zero2),
            pl.BlockSpec((64, 128), zero2),
            pl.BlockSpec((1, 128), zero2),
        ],
        out_specs=pl.BlockSpec((Q4_TILE, 128), lambda g: (g, I32(0))),
        out_shape=jax.ShapeDtypeStruct((N_BATCH * M, 128), F32),
    )(edges, pos_dst, w1p, b1, w2, b2, w3, b3)


def kernel(x, pos, batch, p0, p1, p2, p3, p4, p5):
    x = x.astype(F32)
    pos = pos.astype(F32)
    pos_dst = _sample_positions(pos)
    posTB = jnp.transpose(pos.reshape(N_BATCH, N, 3), (0, 2, 1))
    col = _knn_call(pos_dst, posTB).reshape(-1)
    feats = jnp.concatenate([x.T, pos.T], axis=0).reshape(-1)
    edges = _sc_gather(feats, col).reshape(N_EDGE, D_TBL)
    out = _mlp_call(edges, pos_dst, p0.astype(F32),
                    p1.reshape(1, 64).astype(F32), p2.astype(F32),
                    p3.reshape(1, 64).astype(F32), p4.astype(F32),
                    p5.reshape(1, 128).astype(F32))
    batch_dst = batch.reshape(N_BATCH, N)[:, :M].reshape(-1)
    return out, pos_dst, batch_dst


# speedup vs baseline: 1.2226x; 1.0862x over previous
import functools
import jax
import jax.numpy as jnp
from jax import lax
from jax.experimental import pallas as pl
from jax.experimental.pallas import tpu as pltpu
from jax.experimental.pallas import tpu_sc as plsc

N_PTS = 16384
N_BATCH = 4
N = N_PTS // N_BATCH
M = N // 4
K = 32
F32 = jnp.float32
I32 = jnp.int32


def _two_sum(a, b):
    s = a + b
    bb = s - a
    err = (a - (s - bb)) + (b - bb)
    return s, err


def _fast_two_sum(a, b):
    s = a + b
    return s, b - (s - a)


def _split(a):
    c = a * F32(4097.0)
    hi = c - (c - a)
    return hi, a - hi


def _two_prod(a, b):
    p = a * b
    ahi, alo = _split(a)
    bhi, blo = _split(b)
    err = ((ahi * bhi - p) + ahi * blo + alo * bhi) + alo * blo
    return p, err


def _dd_add(ah, al, bh, bl):
    s, e = _two_sum(ah, bh)
    e = e + (al + bl)
    return _fast_two_sum(s, e)


def _dd_sqdiff(a, b):
    dh, dl = _two_sum(a, -b)
    p = dh * dh
    hi, lo = _split(dh)
    pe = ((hi * hi - p) + F32(2.0) * (hi * lo)) + lo * lo
    pe = pe + F32(2.0) * (dh * dl)
    return _fast_two_sum(p, pe)


def _dd_sqdist(px, py, pz, qx, qy, qz):
    sxh, sxl = _dd_sqdiff(px, qx)
    syh, syl = _dd_sqdiff(py, qy)
    szh, szl = _dd_sqdiff(pz, qz)
    h, l = _dd_add(sxh, sxl, syh, syl)
    return _dd_add(h, l, szh, szl)


R2 = 2 * N_BATCH
NH = N // 2


def _fps_body(posT_ref, out_ref):
    px = posT_ref[0]
    py = posT_ref[1]
    pz = posT_ref[2]
    iota = lax.broadcasted_iota(I32, (N_BATCH, N), 1)

    def store_row(i, qx, qy, qz):
        out_ref[0, pl.ds(i, 1), :] = qx[:, 0][None, :]
        out_ref[1, pl.ds(i, 1), :] = qy[:, 0][None, :]
        out_ref[2, pl.ds(i, 1), :] = qz[:, 0][None, :]

    qx = px[:, 0:1]
    qy = py[:, 0:1]
    qz = pz[:, 0:1]
    store_row(0, qx, qy, qz)
    dh, dl = _dd_sqdist(px, py, pz, qx, qy, qz)

    def body(i, carry):
        dh, dl = carry
        mh = jnp.max(dh, axis=1, keepdims=True)
        eqh = dh == mh
        ml = jnp.max(jnp.where(eqh, dl, -jnp.inf), axis=1, keepdims=True)
        cand = eqh & (dl == ml)
        candf = cand.astype(F32)
        cnt = jnp.sum(candf, axis=1, keepdims=True)
        qxf = jnp.sum(px * candf, axis=1, keepdims=True)
        qyf = jnp.sum(py * candf, axis=1, keepdims=True)
        qzf = jnp.sum(pz * candf, axis=1, keepdims=True)

        def tie_break(_):
            j = jnp.min(jnp.where(cand, iota, I32(N)), axis=1, keepdims=True)
            msk = (iota == j).astype(F32)
            return (jnp.sum(px * msk, axis=1, keepdims=True),
                    jnp.sum(py * msk, axis=1, keepdims=True),
                    jnp.sum(pz * msk, axis=1, keepdims=True))

        qx, qy, qz = lax.cond(jnp.max(cnt) > F32(1.0), tie_break,
                              lambda _: (qxf, qyf, qzf), None)
        store_row(i, qx, qy, qz)
        nh, nl = _dd_sqdist(px, py, pz, qx, qy, qz)
        take = (nh < dh) | ((nh == dh) & (nl < dl))
        return jnp.where(take, nh, dh), jnp.where(take, nl, dl)

    lax.fori_loop(1, M, body, (dh, dl), unroll=2)


def _fps_call(posT):
    return pl.pallas_call(
        _fps_body,
        out_shape=jax.ShapeDtypeStruct((3, M, N_BATCH), F32),
    )(posT)


def _sample_positions(pos):
    posT = jnp.transpose(pos.reshape(N_BATCH, N, 3), (2, 0, 1))
    pd = _fps_call(posT)
    return jnp.transpose(pd, (2, 1, 0)).reshape(N_BATCH * M, 3)


Q_TILE = 256


def _knn_body(q_ref, posT_ref, col_ref):
    b = pl.program_id(0)
    q = q_ref[...]
    qx = q[:, 0:1]
    qy = q[:, 1:2]
    qz = q[:, 2:3]
    s = posT_ref[0]
    sx = s[0:1, :]
    sy = s[1:2, :]
    sz = s[2:3, :]
    dx = qx - sx
    dy = qy - sy
    dz = qz - sz
    d = dx * dx
    d = d + dy * dy
    d = d + dz * dz
    iota = lax.broadcasted_iota(I32, (Q_TILE, N), 1)
    inf = F32(jnp.inf)
    for t in range(K):
        m = jnp.min(d, axis=1, keepdims=True)
        sel = d == m
        j = jnp.min(jnp.where(sel, iota, I32(N)), axis=1, keepdims=True)
        col_ref[:, t:t + 1] = j
        d = jnp.where(iota == j, inf, d)


def _knn_call(pos_dst, posTB):
    return pl.pallas_call(
        _knn_body,
        grid=(N_BATCH, M // Q_TILE),
        in_specs=[
            pl.BlockSpec((Q_TILE, 3), lambda b, t: (b * (M // Q_TILE) + t, I32(0))),
            pl.BlockSpec((1, 3, N), lambda b, t: (b, I32(0), I32(0))),
        ],
        out_specs=pl.BlockSpec((Q_TILE, K), lambda b, t: (b * (M // Q_TILE) + t, I32(0))),
        out_shape=jax.ShapeDtypeStruct((N_BATCH * M, K), I32),
    )(pos_dst, posTB)


D_TBL = 8
N_EDGE = N_BATCH * M * K
HALF = 2048


def _sc_gather(feats, lcol):
    info = plsc.get_sparse_core_info()
    nc, ns = info.num_cores, info.num_subcores
    nw = nc * ns
    e_per_w = N_EDGE // nw

    @functools.partial(
        pl.kernel,
        mesh=plsc.VectorSubcoreMesh(core_axis_name="c", subcore_axis_name="s"),
        out_type=jax.ShapeDtypeStruct((N_EDGE * D_TBL,), F32),
        compiler_params=pltpu.CompilerParams(needs_layout_passes=False),
        scratch_types=[
            pltpu.VMEM((6 * N,), F32),
            pltpu.VMEM((e_per_w,), I32),
            pltpu.VMEM((HALF * D_TBL,), F32),
        ],
    )
    def gather_k(feats_hbm, lcol_hbm, out_hbm, tbl_v, lidx_v, rows_v):
        wid = lax.axis_index("s") * nc + lax.axis_index("c")
        b = wid // (nw // N_BATCH)
        e0 = wid * e_per_w
        for c in range(6):
            pltpu.sync_copy(feats_hbm.at[pl.ds(c * N_PTS + b * N, N)],
                            tbl_v.at[pl.ds(c * N, N)])
        pltpu.sync_copy(lcol_hbm.at[pl.ds(e0, e_per_w)], lidx_v)
        lane = lax.iota(I32, 16)

        def do_half(half, _):
            def grp(g, _):
                iv = lidx_v[pl.ds(half * I32(HALF) + g * I32(16), 16)]
                rows = (g * I32(16) + lane) * I32(D_TBL)
                for c in range(6):
                    v = plsc.load_gather(tbl_v, [iv + I32(c * N)])
                    plsc.store_scatter(rows_v, [rows + I32(c)], v)
                return I32(0)
            lax.fori_loop(I32(0), I32(HALF // 16), grp, I32(0))
            pltpu.sync_copy(
                rows_v,
                out_hbm.at[pl.ds((e0 + half * I32(HALF)) * I32(D_TBL), HALF * D_TBL)])
            return I32(0)

        lax.fori_loop(I32(0), I32(e_per_w // HALF), do_half, I32(0))

    return gather_k(feats, lcol)


E_TILE = 2048
Q4_TILE = E_TILE // K


def _mlp_body(e_ref, q_ref, w1_ref, b1_ref, w2_ref, b2_ref, w3_ref, b3_ref, out_ref):
    hi = lax.Precision.HIGHEST
    t = e_ref[...][:, 0:6]
    h = jnp.dot(t, w1_ref[...], precision=hi, preferred_element_type=F32)
    pq = q_ref[...]
    c = jnp.dot(pq, w1_ref[3:6, :], precision=hi, preferred_element_type=F32)
    h3 = h.reshape(Q4_TILE, K, 64) + (b1_ref[...] - c)[:, None, :]
    h = jnp.maximum(h3, F32(0.0)).reshape(E_TILE, 64)
    h = jnp.dot(h, w2_ref[...], precision=hi, preferred_element_type=F32) + b2_ref[...]
    h = jnp.maximum(h, F32(0.0))
    h = jnp.dot(h, w3_ref[...], precision=hi, preferred_element_type=F32) + b3_ref[...]
    out_ref[...] = jnp.max(h.reshape(Q4_TILE, K, 128), axis=1)


def _mlp_call(edges, pos_dst, w1p, b1, w2, b2, w3, b3):
    n_tile = N_EDGE // E_TILE
    zero2 = lambda g: (I32(0), I32(0))
    return pl.pallas_call(
        _mlp_body,
        grid=(n_tile,),
        in_specs=[
            pl.BlockSpec((E_TILE, D_TBL), lambda g: (g, I32(0))),
            pl.BlockSpec((Q4_TILE, 3), lambda g: (g, I32(0))),
            pl.BlockSpec((6, 64), zero2),
            pl.BlockSpec((1, 64), zero2),
            pl.BlockSpec((64, 64), zero2),
            pl.BlockSpec((1, 64), zero2),
            pl.BlockSpec((64, 128), zero2),
            pl.BlockSpec((1, 128), zero2),
        ],
        out_specs=pl.BlockSpec((Q4_TILE, 128), lambda g: (g, I32(0))),
        out_shape=jax.ShapeDtypeStruct((N_BATCH * M, 128), F32),
    )(edges, pos_dst, w1p, b1, w2, b2, w3, b3)


def kernel(x, pos, batch, p0, p1, p2, p3, p4, p5):
    x = x.astype(F32)
    pos = pos.astype(F32)
    pos_dst = _sample_positions(pos)
    posTB = jnp.transpose(pos.reshape(N_BATCH, N, 3), (0, 2, 1))
    col = _knn_call(pos_dst, posTB).reshape(-1)
    feats = jnp.concatenate([x.T, pos.T], axis=0).reshape(-1)
    edges = _sc_gather(feats, col).reshape(N_EDGE, D_TBL)
    out = _mlp_call(edges, pos_dst, p0.astype(F32),
                    p1.reshape(1, 64).astype(F32), p2.astype(F32),
                    p3.reshape(1, 64).astype(F32), p4.astype(F32),
                    p5.reshape(1, 128).astype(F32))
    batch_dst = batch.reshape(N_BATCH, N)[:, :M].reshape(-1)
    return out, pos_dst, batch_dst


# speedup vs baseline: 1.3126x; 1.0737x over previous
import functools
import jax
import jax.numpy as jnp
from jax import lax
from jax.experimental import pallas as pl
from jax.experimental.pallas import tpu as pltpu
from jax.experimental.pallas import tpu_sc as plsc

N_PTS = 16384
N_BATCH = 4
N = N_PTS // N_BATCH
M = N // 4
K = 32
F32 = jnp.float32
I32 = jnp.int32


def _two_sum(a, b):
    s = a + b
    bb = s - a
    err = (a - (s - bb)) + (b - bb)
    return s, err


def _fast_two_sum(a, b):
    s = a + b
    return s, b - (s - a)


def _split(a):
    c = a * F32(4097.0)
    hi = c - (c - a)
    return hi, a - hi


def _two_prod(a, b):
    p = a * b
    ahi, alo = _split(a)
    bhi, blo = _split(b)
    err = ((ahi * bhi - p) + ahi * blo + alo * bhi) + alo * blo
    return p, err


def _dd_add(ah, al, bh, bl):
    s, e = _two_sum(ah, bh)
    e = e + (al + bl)
    return _fast_two_sum(s, e)


def _dd_sqdiff(a, b):
    dh, dl = _two_sum(a, -b)
    p = dh * dh
    hi, lo = _split(dh)
    pe = ((hi * hi - p) + F32(2.0) * (hi * lo)) + lo * lo
    pe = pe + F32(2.0) * (dh * dl)
    return _fast_two_sum(p, pe)


def _dd_sqdist(px, py, pz, qx, qy, qz):
    sxh, sxl = _dd_sqdiff(px, qx)
    syh, syl = _dd_sqdiff(py, qy)
    szh, szl = _dd_sqdiff(pz, qz)
    h, l = _dd_add(sxh, sxl, syh, syl)
    return _dd_add(h, l, szh, szl)


R2 = 2 * N_BATCH
NH = N // 2


def _fps_body(posT_ref, out_ref):
    px = posT_ref[0]
    py = posT_ref[1]
    pz = posT_ref[2]
    iota = lax.broadcasted_iota(I32, (N_BATCH, N), 1)

    def store_row(i, qx, qy, qz):
        out_ref[0, pl.ds(i, 1), :] = qx[:, 0][None, :]
        out_ref[1, pl.ds(i, 1), :] = qy[:, 0][None, :]
        out_ref[2, pl.ds(i, 1), :] = qz[:, 0][None, :]

    qx = px[:, 0:1]
    qy = py[:, 0:1]
    qz = pz[:, 0:1]
    store_row(0, qx, qy, qz)
    dh, dl = _dd_sqdist(px, py, pz, qx, qy, qz)

    def body(i, carry):
        dh, dl = carry
        mh = jnp.max(dh, axis=1, keepdims=True)
        eqh = dh == mh
        eqf = eqh.astype(F32)
        cnt = jnp.sum(eqf, axis=1, keepdims=True)
        qxf = jnp.sum(px * eqf, axis=1, keepdims=True)
        qyf = jnp.sum(py * eqf, axis=1, keepdims=True)
        qzf = jnp.sum(pz * eqf, axis=1, keepdims=True)

        def tie_break(_):
            ml = jnp.max(jnp.where(eqh, dl, -jnp.inf), axis=1, keepdims=True)
            cand = eqh & (dl == ml)
            j = jnp.min(jnp.where(cand, iota, I32(N)), axis=1, keepdims=True)
            msk = (iota == j).astype(F32)
            return (jnp.sum(px * msk, axis=1, keepdims=True),
                    jnp.sum(py * msk, axis=1, keepdims=True),
                    jnp.sum(pz * msk, axis=1, keepdims=True))

        qx, qy, qz = lax.cond(jnp.max(cnt) > F32(1.0), tie_break,
                              lambda _: (qxf, qyf, qzf), None)
        store_row(i, qx, qy, qz)
        nh, nl = _dd_sqdist(px, py, pz, qx, qy, qz)
        take = (nh < dh) | ((nh == dh) & (nl < dl))
        return jnp.where(take, nh, dh), jnp.where(take, nl, dl)

    lax.fori_loop(1, M, body, (dh, dl), unroll=4)


def _fps_call(posT):
    return pl.pallas_call(
        _fps_body,
        out_shape=jax.ShapeDtypeStruct((3, M, N_BATCH), F32),
    )(posT)


def _sample_positions(pos):
    posT = jnp.transpose(pos.reshape(N_BATCH, N, 3), (2, 0, 1))
    pd = _fps_call(posT)
    return jnp.transpose(pd, (2, 1, 0)).reshape(N_BATCH * M, 3)


Q_TILE = 256


def _knn_body(q_ref, posT_ref, col_ref):
    b = pl.program_id(0)
    q = q_ref[...]
    qx = q[:, 0:1]
    qy = q[:, 1:2]
    qz = q[:, 2:3]
    s = posT_ref[0]
    sx = s[0:1, :]
    sy = s[1:2, :]
    sz = s[2:3, :]
    dx = qx - sx
    dy = qy - sy
    dz = qz - sz
    d = dx * dx
    d = d + dy * dy
    d = d + dz * dz
    iota = lax.broadcasted_iota(I32, (Q_TILE, N), 1)
    inf = F32(jnp.inf)
    for t in range(K):
        m = jnp.min(d, axis=1, keepdims=True)
        sel = d == m
        j = jnp.min(jnp.where(sel, iota, I32(N)), axis=1, keepdims=True)
        col_ref[:, t:t + 1] = j
        d = jnp.where(iota == j, inf, d)


def _knn_call(pos_dst, posTB):
    return pl.pallas_call(
        _knn_body,
        grid=(N_BATCH, M // Q_TILE),
        in_specs=[
            pl.BlockSpec((Q_TILE, 3), lambda b, t: (b * (M // Q_TILE) + t, I32(0))),
            pl.BlockSpec((1, 3, N), lambda b, t: (b, I32(0), I32(0))),
        ],
        out_specs=pl.BlockSpec((Q_TILE, K), lambda b, t: (b * (M // Q_TILE) + t, I32(0))),
        out_shape=jax.ShapeDtypeStruct((N_BATCH * M, K), I32),
    )(pos_dst, posTB)


D_TBL = 8
N_EDGE = N_BATCH * M * K
HALF = 2048


def _sc_gather(feats, lcol):
    info = plsc.get_sparse_core_info()
    nc, ns = info.num_cores, info.num_subcores
    nw = nc * ns
    e_per_w = N_EDGE // nw

    @functools.partial(
        pl.kernel,
        mesh=plsc.VectorSubcoreMesh(core_axis_name="c", subcore_axis_name="s"),
        out_type=jax.ShapeDtypeStruct((N_EDGE * D_TBL,), F32),
        compiler_params=pltpu.CompilerParams(needs_layout_passes=False),
        scratch_types=[
            pltpu.VMEM((6 * N,), F32),
            pltpu.VMEM((e_per_w,), I32),
            pltpu.VMEM((HALF * D_TBL,), F32),
        ],
    )
    def gather_k(feats_hbm, lcol_hbm, out_hbm, tbl_v, lidx_v, rows_v):
        wid = lax.axis_index("s") * nc + lax.axis_index("c")
        b = wid // (nw // N_BATCH)
        e0 = wid * e_per_w
        for c in range(6):
            pltpu.sync_copy(feats_hbm.at[pl.ds(c * N_PTS + b * N, N)],
                            tbl_v.at[pl.ds(c * N, N)])
        pltpu.sync_copy(lcol_hbm.at[pl.ds(e0, e_per_w)], lidx_v)
        lane = lax.iota(I32, 16)

        def do_half(half, _):
            def grp(g, _):
                iv = lidx_v[pl.ds(half * I32(HALF) + g * I32(16), 16)]
                rows = (g * I32(16) + lane) * I32(D_TBL)
                for c in range(6):
                    v = plsc.load_gather(tbl_v, [iv + I32(c * N)])
                    plsc.store_scatter(rows_v, [rows + I32(c)], v)
                return I32(0)
            lax.fori_loop(I32(0), I32(HALF // 16), grp, I32(0))
            pltpu.sync_copy(
                rows_v,
                out_hbm.at[pl.ds((e0 + half * I32(HALF)) * I32(D_TBL), HALF * D_TBL)])
            return I32(0)

        lax.fori_loop(I32(0), I32(e_per_w // HALF), do_half, I32(0))

    return gather_k(feats, lcol)


E_TILE = 2048
Q4_TILE = E_TILE // K


def _mlp_body(e_ref, q_ref, w1_ref, b1_ref, w2_ref, b2_ref, w3_ref, b3_ref, out_ref):
    hi = lax.Precision.HIGHEST
    t = e_ref[...][:, 0:6]
    h = jnp.dot(t, w1_ref[...], precision=hi, preferred_element_type=F32)
    pq = q_ref[...]
    c = jnp.dot(pq, w1_ref[3:6, :], precision=hi, preferred_element_type=F32)
    h3 = h.reshape(Q4_TILE, K, 64) + (b1_ref[...] - c)[:, None, :]
    h = jnp.maximum(h3, F32(0.0)).reshape(E_TILE, 64)
    h = jnp.dot(h, w2_ref[...], precision=hi, preferred_element_type=F32) + b2_ref[...]
    h = jnp.maximum(h, F32(0.0))
    h = jnp.dot(h, w3_ref[...], precision=hi, preferred_element_type=F32) + b3_ref[...]
    out_ref[...] = jnp.max(h.reshape(Q4_TILE, K, 128), axis=1)


def _mlp_call(edges, pos_dst, w1p, b1, w2, b2, w3, b3):
    n_tile = N_EDGE // E_TILE
    zero2 = lambda g: (I32(0), I32(0))
    return pl.pallas_call(
        _mlp_body,
        grid=(n_tile,),
        in_specs=[
            pl.BlockSpec((E_TILE, D_TBL), lambda g: (g, I32(0))),
            pl.BlockSpec((Q4_TILE, 3), lambda g: (g, I32(0))),
            pl.BlockSpec((6, 64), zero2),
            pl.BlockSpec((1, 64), zero2),
            pl.BlockSpec((64, 64), zero2),
            pl.BlockSpec((1, 64), zero2),
            pl.BlockSpec((64, 128), zero2),
            pl.BlockSpec((1, 128), zero2),
        ],
        out_specs=pl.BlockSpec((Q4_TILE, 128), lambda g: (g, I32(0))),
        out_shape=jax.ShapeDtypeStruct((N_BATCH * M, 128), F32),
    )(edges, pos_dst, w1p, b1, w2, b2, w3, b3)


def kernel(x, pos, batch, p0, p1, p2, p3, p4, p5):
    x = x.astype(F32)
    pos = pos.astype(F32)
    pos_dst = _sample_positions(pos)
    posTB = jnp.transpose(pos.reshape(N_BATCH, N, 3), (0, 2, 1))
    col = _knn_call(pos_dst, posTB).reshape(-1)
    feats = jnp.concatenate([x.T, pos.T], axis=0).reshape(-1)
    edges = _sc_gather(feats, col).reshape(N_EDGE, D_TBL)
    out = _mlp_call(edges, pos_dst, p0.astype(F32),
                    p1.reshape(1, 64).astype(F32), p2.astype(F32),
                    p3.reshape(1, 64).astype(F32), p4.astype(F32),
                    p5.reshape(1, 128).astype(F32))
    batch_dst = batch.reshape(N_BATCH, N)[:, :M].reshape(-1)
    return out, pos_dst, batch_dst
